# baseline (speedup 1.0000x reference)
import functools
import jax
import jax.numpy as jnp
from jax import lax
from jax.experimental import pallas as pl
from jax.experimental.pallas import tpu as pltpu, tpu_sc as plsc


def _make_sc_kernel(b, d, h, w):
    mesh = plsc.VectorSubcoreMesh(core_axis_name="c", subcore_axis_name="s")
    n_ch = 16
    hw = h * w
    bsz = 2 * d * hw
    blk = n_ch * hw

    @functools.partial(
        pl.kernel,
        out_type=jax.ShapeDtypeStruct((b * bsz,), jnp.float32),
        mesh=mesh,
        scratch_types=[
            pltpu.VMEM((2 * h, d), jnp.float32),
            pltpu.VMEM((blk,), jnp.float32),
            pltpu.SemaphoreType.DMA,
        ],
        compiler_params=pltpu.CompilerParams(needs_layout_passes=False),
    )
    def k(row_hbm, col_hbm, out_hbm, t, buf, sem):
        nc = 2
        wid = lax.axis_index("s") * nc + lax.axis_index("c")
        wid16 = lax.rem(wid, 16)
        is_row = wid >= 16
        c0 = n_ch * wid16

        pltpu.sync_copy(col_hbm.at[pl.ds(0, h)], t.at[pl.ds(0, h)])
        pltpu.sync_copy(row_hbm.at[pl.ds(0, h)], t.at[pl.ds(h, h)])

        iota = lax.broadcasted_iota(jnp.int32, (16,), 0)
        ccvs = [jnp.full((16,), c0 + cc, jnp.int32) for cc in range(n_ch)]

        for i in range(h):
            rows_row = jnp.full((16,), h + i, jnp.int32)
            for jb in range(w // 16):
                rows = jnp.where(is_row, rows_row, iota + 16 * jb)
                for cc in range(n_ch):
                    v = plsc.load_gather(t, [rows, ccvs[cc]])
                    buf[pl.ds(cc * hw + i * w + 16 * jb, 16)] = v

        out_c = jnp.where(is_row, d + c0, c0)
        base = pl.multiple_of(out_c * hw, hw)
        descs = [
            pltpu.async_copy(buf, out_hbm.at[pl.ds(bi * bsz + base, blk)], sem)
            for bi in range(b)
        ]
        for de in descs:
            de.wait()

    return k


def kernel(x, row_embed, col_embed):
    b = x.shape[0]
    h, w = x.shape[-2], x.shape[-1]
    d = col_embed.shape[-1]
    out = _make_sc_kernel(b, d, h, w)(row_embed, col_embed)
    return out.reshape(b, 2 * d, h, w)


# speedup vs baseline: 15.6946x; 15.6946x over previous
import jax
import jax.numpy as jnp
from jax.experimental import pallas as pl
from jax.experimental.pallas import tpu as pltpu


def _make_body(b, d, h, w):
    hw = h * w

    def body(row_ref, col_ref, out_ref, scratch, sems):
        col = col_ref[0:w, :]
        row = row_ref[0:h, :]
        xp = jnp.broadcast_to(col[None, :, :], (h, w, d)).reshape(hw, d)
        yp = jnp.broadcast_to(row[:, None, :], (h, w, d)).reshape(hw, d)
        scratch[:, 0:d] = xp
        scratch[:, d:2 * d] = yp
        copies = [
            pltpu.make_async_copy(scratch, out_ref.at[i], sems.at[i])
            for i in range(b)
        ]
        for c in copies:
            c.start()
        for c in copies:
            c.wait()
    return body


def kernel(x, row_embed, col_embed):
    b = x.shape[0]
    h, w = x.shape[-2], x.shape[-1]
    d = col_embed.shape[-1]
    hw = h * w
    out_phys = pl.pallas_call(
        _make_body(b, d, h, w),
        in_specs=[
            pl.BlockSpec(memory_space=pltpu.VMEM),
            pl.BlockSpec(memory_space=pltpu.VMEM),
        ],
        out_specs=pl.BlockSpec(memory_space=pltpu.MemorySpace.HBM),
        out_shape=jax.ShapeDtypeStruct((b, hw, 2 * d), jnp.float32),
        scratch_shapes=[
            pltpu.VMEM((hw, 2 * d), jnp.float32),
            pltpu.SemaphoreType.DMA((b,)),
        ],
    )(row_embed, col_embed)
    return out_phys.reshape(b, h, w, 2 * d).transpose(0, 3, 1, 2)
